# baseline (speedup 1.0000x reference)
import jax
import jax.numpy as jnp
from jax.experimental import pallas as pl

N = 50000
G = 64


def _edge_conv(x, edge_index, Wa, ba, Wb, bb):
    src = edge_index[0]
    dst = edge_index[1]
    xi = x[dst]
    xj = x[src]
    m = jnp.concatenate([xi, xj - xi], axis=1)
    h = jnp.maximum(m @ Wa + ba, 0.0) @ Wb + bb
    agg = jax.ops.segment_max(h, dst, num_segments=N)
    return jnp.where(jnp.isneginf(agg), 0.0, agg)


def _head_body(pooled_ref, W5_ref, b5_ref, W6_ref, b6_ref, out_ref):
    h = jnp.maximum(pooled_ref[...] @ W5_ref[...] + b5_ref[...][None, :], 0.0)
    o = h @ W6_ref[...] + b6_ref[...][None, :]
    m = jnp.max(o, axis=1, keepdims=True)
    lse = m + jnp.log(jnp.sum(jnp.exp(o - m), axis=1, keepdims=True))
    out_ref[...] = o - lse


def kernel(x, edge_index, batch, W1, b1, W2, b2, W3, b3, W4, b4, W5, b5, W6, b6):
    h = jnp.maximum(_edge_conv(x, edge_index, W1, b1, W2, b2), 0.0)
    h = jnp.maximum(_edge_conv(h, edge_index, W3, b3, W4, b4), 0.0)
    pooled = jax.ops.segment_max(h, batch, num_segments=G)
    pooled = jnp.where(jnp.isneginf(pooled), 0.0, pooled)
    out = pl.pallas_call(
        _head_body,
        out_shape=jax.ShapeDtypeStruct((G, 1), jnp.float32),
    )(pooled, W5, b5, W6, b6)
    return out


# speedup vs baseline: 1.8093x; 1.8093x over previous
import functools

import jax
import jax.numpy as jnp
from jax import lax
from jax.experimental import pallas as pl
from jax.experimental.pallas import tpu as pltpu
from jax.experimental.pallas import tpu_sc as plsc

NN = 50000
EE = 800000
GG = 64
NC, NS, LN = 2, 16, 16
NW = NC * NS
EW = EE // NW
RB = 784
NB = 64
GST = 26624
LCAP = 25448
WIN = 1024
GBATCH = 64
NEGINF = float("-inf")


def _mesh():
    return plsc.VectorSubcoreMesh(core_axis_name="c", subcore_axis_name="s",
                                  num_cores=NC, num_subcores=NS)


def _wid():
    return lax.axis_index("c") * NS + lax.axis_index("s")


def _iota16():
    return lax.broadcasted_iota(jnp.int32, (16,), 0)


def _m8(v):
    return pl.multiple_of(v, 8)


def _scal(ref, i):
    base = pl.multiple_of((i >> 4) << 4, 16)
    v = ref[pl.ds(base, 16)]
    sel = jnp.where(_iota16() == (i - base), v, jnp.int32(-2147483647))
    return jnp.max(sel, axis=0)


def _lane_bcast(v, k):
    kv = jnp.zeros((16,), jnp.int32) + k
    return jnp.take_along_axis(v, kv, axis=0, mode="promise_in_bounds")


def _edge_body(C, dst_h, src_h, A_h, B_h, Z_h, idxd, idxs, rows, rows2, s1, s2):
    w = _wid()
    base = w * EW

    def chunk(off, n):
        pltpu.sync_copy(dst_h.at[pl.ds(off, n)], idxd.at[pl.ds(0, n)])
        pltpu.sync_copy(src_h.at[pl.ds(off, n)], idxs.at[pl.ds(0, n)])
        ca = pltpu.async_copy(A_h.at[idxd.at[pl.ds(0, n)]], rows.at[pl.ds(0, n)], s1)
        cb = pltpu.async_copy(B_h.at[idxs.at[pl.ds(0, n)]], rows2.at[pl.ds(0, n)], s2)
        ca.wait()
        cb.wait()

        def add_body(r, _):
            for j in range(C // 16):
                rows[r, pl.ds(j * 16, 16)] = (
                    rows[r, pl.ds(j * 16, 16)] + rows2[r, pl.ds(j * 16, 16)]
                )
            return 0

        lax.fori_loop(0, n, add_body, 0)
        pltpu.sync_copy(rows.at[pl.ds(0, n)], Z_h.at[pl.ds(off, n)])

    lax.fori_loop(0, 195, lambda i, _: (chunk(_m8(base + i * 128), 128), 0)[1], 0)
    chunk(_m8(base + 195 * 128), 40)


def _edge_call(C, dst, src, A, B):
    body = functools.partial(_edge_body, C)
    return pl.kernel(
        body,
        out_type=jax.ShapeDtypeStruct((EE, C), jnp.float32),
        mesh=_mesh(),
        compiler_params=pltpu.CompilerParams(needs_layout_passes=False),
        scratch_types=[
            pltpu.VMEM((128,), jnp.int32),
            pltpu.VMEM((128,), jnp.int32),
            pltpu.VMEM((128, C), jnp.float32),
            pltpu.VMEM((128, C), jnp.float32),
            pltpu.SemaphoreType.DMA,
            pltpu.SemaphoreType.DMA,
        ],
        name=f"sc_edge_{C}",
    )(dst, src, A, B)


def _bin_body(dstp_h, bid_h, bdst_h, hist_h, offs_h, dstv, lid, ldst, hist, offs, rcnt):
    w = _wid()
    base = w * EW
    iota = _iota16()
    rcal, _ = plsc.scan_count(iota)
    bias = jnp.max(rcal, axis=0)

    for q in range(4):
        hist[pl.ds(q * 16, 16)] = jnp.zeros((16,), jnp.int32)

    def scan(pass2):
        def do_chunk(coff, sz, nv, tail):
            pltpu.sync_copy(dstp_h.at[pl.ds(_m8(base + coff), sz)], dstv.at[pl.ds(0, sz)])

            def vreg(j, _):
                d = dstv[pl.ds(j * 16, 16)]
                b = lax.div(d, jnp.int32(RB))
                if tail:
                    valid = (coff + j * 16 + iota) < EW
                else:
                    valid = jnp.full((16,), True)
                rank, lastm = plsc.scan_count(b, mask=valid)
                r0 = rank - bias
                if pass2:
                    cur = plsc.load_gather(rcnt, [b])
                    pos = cur + r0
                    eid = base + coff + j * 16 + iota
                    plsc.store_scatter(lid, [pos], eid, mask=valid)
                    plsc.store_scatter(ldst, [pos], d, mask=valid)
                    plsc.store_scatter(rcnt, [b], cur + r0 + 1, mask=lastm & valid)
                else:
                    cur = plsc.load_gather(hist, [b])
                    plsc.store_scatter(hist, [b], cur + r0 + 1, mask=lastm & valid)
                return 0

            lax.fori_loop(0, nv, vreg, 0)

        lax.fori_loop(0, 12, lambda i, _: (do_chunk(i * 2000, 2000, 125, False), 0)[1], 0)
        do_chunk(24000, 1008, 63, True)

    scan(False)

    carry = jnp.int32(0)
    for q in range(4):
        h16 = hist[pl.ds(q * 16, 16)]
        p16 = jnp.bitwise_and(h16 + 7, jnp.int32(-8))
        cs = plsc.cumsum(p16)
        offs[pl.ds(q * 16, 16)] = cs - p16 + carry
        carry = carry + jnp.max(cs, axis=0)
    for q in range(4):
        rcnt[pl.ds(q * 16, 16)] = offs[pl.ds(q * 16, 16)]

    scan(True)

    pltpu.sync_copy(hist, hist_h.at[pl.ds(_m8(w * NB), NB)])
    pltpu.sync_copy(offs, offs_h.at[pl.ds(_m8(w * NB), NB)])
    pltpu.sync_copy(lid, bid_h.at[pl.ds(_m8(w * GST), LCAP)])
    pltpu.sync_copy(ldst, bdst_h.at[pl.ds(_m8(w * GST), LCAP)])


def _bin_call(dstp):
    return pl.kernel(
        _bin_body,
        out_type=(
            jax.ShapeDtypeStruct((NW * GST,), jnp.int32),
            jax.ShapeDtypeStruct((NW * GST,), jnp.int32),
            jax.ShapeDtypeStruct((NW * NB,), jnp.int32),
            jax.ShapeDtypeStruct((NW * NB,), jnp.int32),
        ),
        mesh=_mesh(),
        compiler_params=pltpu.CompilerParams(needs_layout_passes=False),
        scratch_types=[
            pltpu.VMEM((2000,), jnp.int32),
            pltpu.VMEM((LCAP,), jnp.int32),
            pltpu.VMEM((LCAP,), jnp.int32),
            pltpu.VMEM((NB,), jnp.int32),
            pltpu.VMEM((NB,), jnp.int32),
            pltpu.VMEM((NB,), jnp.int32),
        ],
        name="sc_bin",
    )(dstp)


def _process_bin(C, CT, vr, lo, bid_h, bdst_h, H_h, histv, offsv, idw, dstw, hrows, acc, sem):
    iota = _iota16()

    def per_wk(wk, _):
        cnt = _scal(histv, wk * NB + vr)
        off = _scal(offsv, wk * NB + vr)
        gbase = wk * GST + off
        nwin = (cnt + (WIN - 1)) // WIN

        def per_win(cw, _):
            vw = jnp.minimum(cnt - cw * WIN, WIN)
            pltpu.sync_copy(bid_h.at[pl.ds(_m8(gbase + cw * WIN), WIN)], idw)
            pltpu.sync_copy(bdst_h.at[pl.ds(_m8(gbase + cw * WIN), WIN)], dstw)
            safe = _scal(idw, 0)

            def fix(q, _):
                qb = pl.multiple_of(q * 16, 16)
                iv = idw[pl.ds(qb, 16)]
                posv = q * 16 + iota
                idw[pl.ds(qb, 16)] = jnp.where(posv < vw, iv, safe)
                return 0

            lax.fori_loop(0, WIN // 16, fix, 0)
            nb = (vw + (GBATCH - 1)) // GBATCH

            def per_batch(b, _):
                pltpu.async_copy(
                    H_h.at[idw.at[pl.ds(b * GBATCH, GBATCH)]], hrows, sem
                ).wait()
                ne = jnp.minimum(vw - b * GBATCH, GBATCH)

                def per_row(e, _):
                    q = b * GBATCH + e
                    dv = dstw[pl.ds(pl.multiple_of((q >> 4) << 4, 16), 16)]
                    db = _lane_bcast(dv, q - ((q >> 4) << 4))
                    a0 = (db - lo) * C
                    for j in range(C // 16):
                        aj = a0 + (iota + j * 16)
                        cur = plsc.load_gather(acc, [aj])
                        hv = hrows[e, pl.ds(j * 16, 16)]
                        plsc.store_scatter(acc, [aj], jnp.maximum(cur, hv))
                    return 0

                lax.fori_loop(0, ne, per_row, 0)
                return 0

            lax.fori_loop(0, nb, per_batch, 0)
            return 0

        lax.fori_loop(0, nwin, per_win, 0)
        return 0

    lax.fori_loop(0, NW, per_wk, 0)


def _scat1_body(bid_h, bdst_h, hist_h, offs_h, H_h, h1_h,
                histv, offsv, idw, dstw, hrows, acc, sem):
    C = 64
    w = _wid()
    lo = w * (2 * RB)
    pltpu.sync_copy(hist_h, histv)
    pltpu.sync_copy(offs_h, offsv)

    def init(i, _):
        for u in range(8):
            acc[pl.ds(pl.multiple_of(i * 128 + u * 16, 16), 16)] = jnp.full((16,), NEGINF, jnp.float32)
        return 0

    lax.fori_loop(0, (2 * RB * C) // 128, init, 0)
    for sb in range(2):
        _process_bin(C, 128, 2 * w + sb, lo, bid_h, bdst_h, H_h,
                     histv, offsv, idw, dstw, hrows, acc, sem)

    def fin(i, _):
        for u in range(8):
            j = pl.multiple_of(i * 128 + u * 16, 16)
            acc[pl.ds(j, 16)] = jnp.maximum(acc[pl.ds(j, 16)], 0.0)
        return 0

    lax.fori_loop(0, (2 * RB * C) // 128, fin, 0)
    pltpu.sync_copy(acc, h1_h.at[pl.ds(_m8(lo * C), 2 * RB * C)])


def _scat1_call(bid, bdst, hist, offs, H1):
    return pl.kernel(
        _scat1_body,
        out_type=jax.ShapeDtypeStruct((NW * 2 * RB * 64,), jnp.float32),
        mesh=_mesh(),
        compiler_params=pltpu.CompilerParams(needs_layout_passes=False),
        scratch_types=[
            pltpu.VMEM((NW * NB,), jnp.int32),
            pltpu.VMEM((NW * NB,), jnp.int32),
            pltpu.VMEM((WIN,), jnp.int32),
            pltpu.VMEM((WIN,), jnp.int32),
            pltpu.VMEM((GBATCH, 128), jnp.float32),
            pltpu.VMEM((2 * RB * 64,), jnp.float32),
            pltpu.SemaphoreType.DMA,
        ],
        name="sc_scat1",
    )(bid, bdst, hist, offs, H1)


def _scat2_body(bid_h, bdst_h, hist_h, offs_h, H_h, batch_h, part_h,
                histv, offsv, idw, dstw, hrows, acc, batchv, pool, sem):
    C = 128
    w = _wid()
    iota = _iota16()
    pltpu.sync_copy(hist_h, histv)
    pltpu.sync_copy(offs_h, offsv)
    for sb in range(2):
        vr = 2 * w + sb
        lo = vr * RB

        def init(i, _):
            for u in range(8):
                acc[pl.ds(pl.multiple_of(i * 128 + u * 16, 16), 16)] = jnp.full((16,), NEGINF, jnp.float32)
            return 0

        lax.fori_loop(0, (RB * C) // 128, init, 0)
        _process_bin(C, 128, vr, lo, bid_h, bdst_h, H_h,
                     histv, offsv, idw, dstw, hrows, acc, sem)

        pltpu.sync_copy(batch_h.at[pl.ds(_m8(lo), RB)], batchv)

        def pinit(i, _):
            for u in range(8):
                pool[pl.ds(pl.multiple_of(i * 128 + u * 16, 16), 16)] = jnp.full((16,), NEGINF, jnp.float32)
            return 0

        lax.fori_loop(0, (GG * C) // 128, pinit, 0)
        nr = jnp.minimum(RB, NN - lo)

        def pnode(n, _):
            bv = batchv[pl.ds(pl.multiple_of((n >> 4) << 4, 16), 16)]
            bb = _lane_bcast(bv, n - ((n >> 4) << 4))
            a0 = bb * C
            for j in range(C // 16):
                aj = a0 + (iota + j * 16)
                v = jnp.maximum(acc[pl.ds(pl.multiple_of(n * C + j * 16, 16), 16)], 0.0)
                cur = plsc.load_gather(pool, [aj])
                plsc.store_scatter(pool, [aj], jnp.maximum(cur, v))
            return 0

        lax.fori_loop(0, nr, pnode, 0)
        pltpu.sync_copy(pool, part_h.at[pl.ds(_m8(vr * GG * C), GG * C)])


def _scat2_call(bid, bdst, hist, offs, H2, batch_p):
    return pl.kernel(
        _scat2_body,
        out_type=jax.ShapeDtypeStruct((NB * GG * 128,), jnp.float32),
        mesh=_mesh(),
        compiler_params=pltpu.CompilerParams(needs_layout_passes=False),
        scratch_types=[
            pltpu.VMEM((NW * NB,), jnp.int32),
            pltpu.VMEM((NW * NB,), jnp.int32),
            pltpu.VMEM((WIN,), jnp.int32),
            pltpu.VMEM((WIN,), jnp.int32),
            pltpu.VMEM((GBATCH, 128), jnp.float32),
            pltpu.VMEM((RB * 128,), jnp.float32),
            pltpu.VMEM((RB,), jnp.int32),
            pltpu.VMEM((GG * 128,), jnp.float32),
            pltpu.SemaphoreType.DMA,
        ],
        name="sc_scat2",
    )(bid, bdst, hist, offs, H2, batch_p)


def _prep1_body(x_ref, W1_ref, b1_ref, A_ref, B_ref):
    Wt = W1_ref[0:4, :]
    Wb = W1_ref[4:8, :]
    x = x_ref[...]
    za = jnp.zeros((x.shape[0], 64), jnp.float32)
    A_ref[...] = jnp.concatenate(
        [jax.lax.dot_general(x, Wt - Wb, (((1,), (0,)), ((), ())),
                             preferred_element_type=jnp.float32) + b1_ref[...][None, :], za], axis=1)
    B_ref[...] = jnp.concatenate(
        [jax.lax.dot_general(x, Wb, (((1,), (0,)), ((), ())),
                             preferred_element_type=jnp.float32), za], axis=1)


def _prep1_call(x, W1, b1):
    return pl.pallas_call(
        _prep1_body,
        grid=(10,),
        in_specs=[
            pl.BlockSpec((5000, 4), lambda i: (i, 0)),
            pl.BlockSpec((8, 64), lambda i: (0, 0)),
            pl.BlockSpec((64,), lambda i: (0,)),
        ],
        out_specs=[
            pl.BlockSpec((5000, 128), lambda i: (i, 0)),
            pl.BlockSpec((5000, 128), lambda i: (i, 0)),
        ],
        out_shape=[
            jax.ShapeDtypeStruct((NN, 128), jnp.float32),
            jax.ShapeDtypeStruct((NN, 128), jnp.float32),
        ],
        name="tc_prep1",
    )(x, W1, b1)


def _prep2_body(h_ref, W3_ref, b3_ref, A_ref, B_ref):
    Wt = W3_ref[0:64, :]
    Wb = W3_ref[64:128, :]
    h = h_ref[...]
    A_ref[...] = jax.lax.dot_general(h, Wt - Wb, (((1,), (0,)), ((), ())),
                                     preferred_element_type=jnp.float32) + b3_ref[...][None, :]
    B_ref[...] = jax.lax.dot_general(h, Wb, (((1,), (0,)), ((), ())),
                                     preferred_element_type=jnp.float32)


def _prep2_call(h1, W3, b3):
    return pl.pallas_call(
        _prep2_body,
        grid=(10,),
        in_specs=[
            pl.BlockSpec((5000, 64), lambda i: (i, 0)),
            pl.BlockSpec((128, 128), lambda i: (0, 0)),
            pl.BlockSpec((128,), lambda i: (0,)),
        ],
        out_specs=[
            pl.BlockSpec((5000, 128), lambda i: (i, 0)),
            pl.BlockSpec((5000, 128), lambda i: (i, 0)),
        ],
        out_shape=[
            jax.ShapeDtypeStruct((NN, 128), jnp.float32),
            jax.ShapeDtypeStruct((NN, 128), jnp.float32),
        ],
        name="tc_prep2",
    )(h1, W3, b3)


def _mm_body(z_ref, W_ref, b_ref, out_ref):
    z = jnp.maximum(z_ref[...], 0.0)
    out_ref[...] = jax.lax.dot_general(z, W_ref[...], (((1,), (0,)), ((), ())),
                                       preferred_element_type=jnp.float32) + b_ref[...][None, :]


def _mm_call(C, Z, W, b):
    return pl.pallas_call(
        _mm_body,
        grid=(800,),
        in_specs=[
            pl.BlockSpec((1000, C), lambda i: (i, 0)),
            pl.BlockSpec((C, C), lambda i: (0, 0)),
            pl.BlockSpec((C,), lambda i: (0,)),
        ],
        out_specs=pl.BlockSpec((1000, C), lambda i: (i, 0)),
        out_shape=jax.ShapeDtypeStruct((EE, C), jnp.float32),
        name=f"tc_mm_{C}",
    )(Z, W, b)


def _head_body(part_ref, W5_ref, b5_ref, W6_ref, b6_ref, out_ref):
    pm = jnp.max(part_ref[...], axis=0)
    pooled = jnp.maximum(pm, 0.0)
    h = jnp.maximum(
        jax.lax.dot_general(pooled, W5_ref[...], (((1,), (0,)), ((), ())),
                            preferred_element_type=jnp.float32) + b5_ref[...][None, :], 0.0)
    o = jax.lax.dot_general(h, W6_ref[...], (((1,), (0,)), ((), ())),
                            preferred_element_type=jnp.float32) + b6_ref[...][None, :]
    m = jnp.max(o, axis=1, keepdims=True)
    lse = m + jnp.log(jnp.sum(jnp.exp(o - m), axis=1, keepdims=True))
    out_ref[...] = o - lse


def _head_call(part, W5, b5, W6, b6):
    return pl.pallas_call(
        _head_body,
        out_shape=jax.ShapeDtypeStruct((GG, 1), jnp.float32),
        name="tc_head",
    )(part, W5, b5, W6, b6)


def kernel(x, edge_index, batch, W1, b1, W2, b2, W3, b3, W4, b4, W5, b5, W6, b6):
    src = edge_index[0]
    dst = edge_index[1]
    dst_pad = jnp.concatenate([dst, jnp.zeros((16,), jnp.int32)])
    batch_pad = jnp.concatenate([batch, jnp.zeros((NB * RB - NN,), jnp.int32)])

    bid, bdst, hist, offs = _bin_call(dst_pad)

    A1, B1 = _prep1_call(x, W1, b1)
    Z1 = _edge_call(128, dst, src, A1, B1)
    W2p = jnp.zeros((128, 128), jnp.float32).at[:64, :64].set(W2)
    b2p = jnp.zeros((128,), jnp.float32).at[:64].set(b2)
    H1 = _mm_call(128, Z1, W2p, b2p)
    h1f = _scat1_call(bid, bdst, hist, offs, H1)
    h1 = h1f[: NN * 64].reshape(NN, 64)

    A2, B2 = _prep2_call(h1, W3, b3)
    Z2 = _edge_call(128, dst, src, A2, B2)
    H2 = _mm_call(128, Z2, W4, b4)
    part = _scat2_call(bid, bdst, hist, offs, H2, batch_pad)

    part3 = part.reshape(NB, GG, 128)
    return _head_call(part3, W5, b5, W6, b6)


# speedup vs baseline: 2.1124x; 1.1675x over previous
import functools

import jax
import jax.numpy as jnp
from jax import lax
from jax.experimental import pallas as pl
from jax.experimental.pallas import tpu as pltpu
from jax.experimental.pallas import tpu_sc as plsc

NN = 50000
EE = 800000
GG = 64
NC, NS, LN = 2, 16, 16
NW = NC * NS
EW = EE // NW
RB = 784
NB = 64
GST = 26624
LCAP = 25448
WIN = 1024
GBATCH = 64
NEGINF = float("-inf")


def _mesh():
    return plsc.VectorSubcoreMesh(core_axis_name="c", subcore_axis_name="s",
                                  num_cores=NC, num_subcores=NS)


def _wid():
    return lax.axis_index("c") * NS + lax.axis_index("s")


def _iota16():
    return lax.broadcasted_iota(jnp.int32, (16,), 0)


def _m8(v):
    return pl.multiple_of(v, 8)


def _scal(ref, i):
    base = pl.multiple_of((i >> 4) << 4, 16)
    v = ref[pl.ds(base, 16)]
    sel = jnp.where(_iota16() == (i - base), v, jnp.int32(-2147483647))
    return jnp.max(sel, axis=0)


def _lane_bcast(v, k):
    kv = jnp.zeros((16,), jnp.int32) + k
    return jnp.take_along_axis(v, kv, axis=0, mode="promise_in_bounds")


def _edge_body(C, dst_h, src_h, A_h, B_h, Z_h, idxd, idxs, rows, rows2, s1, s2):
    w = _wid()
    base = w * EW

    def chunk(off, n):
        pltpu.sync_copy(dst_h.at[pl.ds(off, n)], idxd.at[pl.ds(0, n)])
        pltpu.sync_copy(src_h.at[pl.ds(off, n)], idxs.at[pl.ds(0, n)])
        ca = pltpu.async_copy(A_h.at[idxd.at[pl.ds(0, n)]], rows.at[pl.ds(0, n)], s1)
        cb = pltpu.async_copy(B_h.at[idxs.at[pl.ds(0, n)]], rows2.at[pl.ds(0, n)], s2)
        ca.wait()
        cb.wait()

        def add_body(r, _):
            for j in range(C // 16):
                rows[r, pl.ds(j * 16, 16)] = (
                    rows[r, pl.ds(j * 16, 16)] + rows2[r, pl.ds(j * 16, 16)]
                )
            return 0

        lax.fori_loop(0, n, add_body, 0)
        pltpu.sync_copy(rows.at[pl.ds(0, n)], Z_h.at[pl.ds(off, n)])

    lax.fori_loop(0, 195, lambda i, _: (chunk(_m8(base + i * 128), 128), 0)[1], 0)
    chunk(_m8(base + 195 * 128), 40)


def _edge_call(C, dst, src, A, B):
    body = functools.partial(_edge_body, C)
    return pl.kernel(
        body,
        out_type=jax.ShapeDtypeStruct((EE, C), jnp.float32),
        mesh=_mesh(),
        compiler_params=pltpu.CompilerParams(needs_layout_passes=False),
        scratch_types=[
            pltpu.VMEM((128,), jnp.int32),
            pltpu.VMEM((128,), jnp.int32),
            pltpu.VMEM((128, C), jnp.float32),
            pltpu.VMEM((128, C), jnp.float32),
            pltpu.SemaphoreType.DMA,
            pltpu.SemaphoreType.DMA,
        ],
        name=f"sc_edge_{C}",
    )(dst, src, A, B)


def _bin_body(dstp_h, bid_h, bdst_h, hist_h, offs_h, dstv, lid, ldst, hist, offs, rcnt):
    w = _wid()
    base = w * EW
    iota = _iota16()
    rcal, _ = plsc.scan_count(iota)
    bias = jnp.max(rcal, axis=0)

    for q in range(4):
        hist[pl.ds(q * 16, 16)] = jnp.zeros((16,), jnp.int32)

    def scan(pass2):
        def do_chunk(coff, sz, nv, tail):
            pltpu.sync_copy(dstp_h.at[pl.ds(_m8(base + coff), sz)], dstv.at[pl.ds(0, sz)])

            def vreg(j, _):
                d = dstv[pl.ds(j * 16, 16)]
                b = lax.div(d, jnp.int32(RB))
                if tail:
                    valid = (coff + j * 16 + iota) < EW
                else:
                    valid = jnp.full((16,), True)
                rank, lastm = plsc.scan_count(b, mask=valid)
                r0 = rank - bias
                if pass2:
                    cur = plsc.load_gather(rcnt, [b])
                    pos = cur + r0
                    eid = base + coff + j * 16 + iota
                    plsc.store_scatter(lid, [pos], eid, mask=valid)
                    plsc.store_scatter(ldst, [pos], d, mask=valid)
                    plsc.store_scatter(rcnt, [b], cur + r0 + 1, mask=lastm & valid)
                else:
                    cur = plsc.load_gather(hist, [b])
                    plsc.store_scatter(hist, [b], cur + r0 + 1, mask=lastm & valid)
                return 0

            lax.fori_loop(0, nv, vreg, 0)

        lax.fori_loop(0, 12, lambda i, _: (do_chunk(i * 2000, 2000, 125, False), 0)[1], 0)
        do_chunk(24000, 1008, 63, True)

    scan(False)

    carry = jnp.int32(0)
    for q in range(4):
        h16 = hist[pl.ds(q * 16, 16)]
        p16 = jnp.bitwise_and(h16 + 7, jnp.int32(-8))
        cs = plsc.cumsum(p16)
        offs[pl.ds(q * 16, 16)] = cs - p16 + carry
        carry = carry + jnp.max(cs, axis=0)
    for q in range(4):
        rcnt[pl.ds(q * 16, 16)] = offs[pl.ds(q * 16, 16)]

    scan(True)

    pltpu.sync_copy(hist, hist_h.at[pl.ds(_m8(w * NB), NB)])
    pltpu.sync_copy(offs, offs_h.at[pl.ds(_m8(w * NB), NB)])
    pltpu.sync_copy(lid, bid_h.at[pl.ds(_m8(w * GST), LCAP)])
    pltpu.sync_copy(ldst, bdst_h.at[pl.ds(_m8(w * GST), LCAP)])


def _bin_call(dstp):
    return pl.kernel(
        _bin_body,
        out_type=(
            jax.ShapeDtypeStruct((NW * GST,), jnp.int32),
            jax.ShapeDtypeStruct((NW * GST,), jnp.int32),
            jax.ShapeDtypeStruct((NW * NB,), jnp.int32),
            jax.ShapeDtypeStruct((NW * NB,), jnp.int32),
        ),
        mesh=_mesh(),
        compiler_params=pltpu.CompilerParams(needs_layout_passes=False),
        scratch_types=[
            pltpu.VMEM((2000,), jnp.int32),
            pltpu.VMEM((LCAP,), jnp.int32),
            pltpu.VMEM((LCAP,), jnp.int32),
            pltpu.VMEM((NB,), jnp.int32),
            pltpu.VMEM((NB,), jnp.int32),
            pltpu.VMEM((NB,), jnp.int32),
        ],
        name="sc_bin",
    )(dstp)


def _process_bin(C, CT, vr, lo, bid_h, bdst_h, H_h, histv, offsv, idw, dstw, hrows, hrows2, acc, sem, sem2):
    iota = _iota16()

    def per_wk(wk, _):
        cnt = _scal(histv, wk * NB + vr)
        off = _scal(offsv, wk * NB + vr)
        gbase = wk * GST + off
        nwin = (cnt + (WIN - 1)) // WIN

        def per_win(cw, _):
            vw = jnp.minimum(cnt - cw * WIN, WIN)
            pltpu.sync_copy(bid_h.at[pl.ds(_m8(gbase + cw * WIN), WIN)], idw)
            pltpu.sync_copy(bdst_h.at[pl.ds(_m8(gbase + cw * WIN), WIN)], dstw)
            safe = _scal(idw, 0)
            safed = _scal(dstw, 0)

            def fix(q, _):
                qb = pl.multiple_of(q * 16, 16)
                posv = q * 16 + iota
                idw[pl.ds(qb, 16)] = jnp.where(posv < vw, idw[pl.ds(qb, 16)], safe)
                dstw[pl.ds(qb, 16)] = jnp.where(posv < vw, dstw[pl.ds(qb, 16)], safed)
                return 0

            lax.fori_loop(0, WIN // 16, fix, 0)
            gb = hrows.shape[0]
            nb = (vw + (gb - 1)) // gb
            pltpu.async_copy(H_h.at[idw.at[pl.ds(0, gb)]], hrows, sem)

            def do_batch(b, hr, sm, hrn, smn):
                gb = hr.shape[0]

                @pl.when(b + 1 < nb)
                def _():
                    pltpu.async_copy(
                        H_h.at[idw.at[pl.ds(_m8((b + 1) * gb), gb)]], hrn, smn
                    )

                pltpu.make_async_copy(H_h.at[idw.at[pl.ds(0, gb)]], hr, sm).wait()
                ne = jnp.minimum(vw - b * gb, gb)

                def per_pair(g, _):
                    for t in range(2):
                        e = g * 2 + t
                        q = b * gb + e
                        dv = dstw[pl.ds(pl.multiple_of((q >> 4) << 4, 16), 16)]
                        db = _lane_bcast(dv, q - ((q >> 4) << 4))
                        a0 = (db - lo) * C
                        for j in range(C // 16):
                            aj = a0 + (iota + j * 16)
                            cur = plsc.load_gather(acc, [aj])
                            hv = hr[e, pl.ds(j * 16, 16)]
                            plsc.store_scatter(acc, [aj], jnp.maximum(cur, hv))
                    return 0

                lax.fori_loop(0, (ne + 1) >> 1, per_pair, 0)

            def per_bpair(gb, _):
                @pl.when(2 * gb < nb)
                def _():
                    do_batch(2 * gb, hrows, sem, hrows2, sem2)

                @pl.when(2 * gb + 1 < nb)
                def _():
                    do_batch(2 * gb + 1, hrows2, sem2, hrows, sem)

                return 0

            lax.fori_loop(0, (nb + 1) >> 1, per_bpair, 0)
            return 0

        lax.fori_loop(0, nwin, per_win, 0)
        return 0

    lax.fori_loop(0, NW, per_wk, 0)


def _scat1_body(bid_h, bdst_h, hist_h, offs_h, H_h, h1_h,
                histv, offsv, idw, dstw, hrows, hrows2, acc, sem, sem2):
    C = 64
    w = _wid()
    lo = w * (2 * RB)
    pltpu.sync_copy(hist_h, histv)
    pltpu.sync_copy(offs_h, offsv)

    def init(i, _):
        for u in range(8):
            acc[pl.ds(pl.multiple_of(i * 128 + u * 16, 16), 16)] = jnp.full((16,), NEGINF, jnp.float32)
        return 0

    lax.fori_loop(0, (2 * RB * C) // 128, init, 0)
    for sb in range(2):
        _process_bin(C, 128, 2 * w + sb, lo, bid_h, bdst_h, H_h,
                     histv, offsv, idw, dstw, hrows, hrows2, acc, sem, sem2)

    def fin(i, _):
        for u in range(8):
            j = pl.multiple_of(i * 128 + u * 16, 16)
            acc[pl.ds(j, 16)] = jnp.maximum(acc[pl.ds(j, 16)], 0.0)
        return 0

    lax.fori_loop(0, (2 * RB * C) // 128, fin, 0)
    pltpu.sync_copy(acc, h1_h.at[pl.ds(_m8(lo * C), 2 * RB * C)])


def _scat1_call(bid, bdst, hist, offs, H1):
    return pl.kernel(
        _scat1_body,
        out_type=jax.ShapeDtypeStruct((NW * 2 * RB * 64,), jnp.float32),
        mesh=_mesh(),
        compiler_params=pltpu.CompilerParams(needs_layout_passes=False),
        scratch_types=[
            pltpu.VMEM((NW * NB,), jnp.int32),
            pltpu.VMEM((NW * NB,), jnp.int32),
            pltpu.VMEM((WIN,), jnp.int32),
            pltpu.VMEM((WIN,), jnp.int32),
            pltpu.VMEM((GBATCH, 128), jnp.float32),
            pltpu.VMEM((GBATCH, 128), jnp.float32),
            pltpu.VMEM((2 * RB * 64,), jnp.float32),
            pltpu.SemaphoreType.DMA,
            pltpu.SemaphoreType.DMA,
        ],
        name="sc_scat1",
    )(bid, bdst, hist, offs, H1)


def _scat2_body(bid_h, bdst_h, hist_h, offs_h, H_h, batch_h, part_h,
                histv, offsv, idw, dstw, hrows, hrows2, acc, batchv, pool, sem, sem2):
    C = 128
    w = _wid()
    iota = _iota16()
    pltpu.sync_copy(hist_h, histv)
    pltpu.sync_copy(offs_h, offsv)
    for sb in range(2):
        vr = 2 * w + sb
        lo = vr * RB

        def init(i, _):
            for u in range(8):
                acc[pl.ds(pl.multiple_of(i * 128 + u * 16, 16), 16)] = jnp.full((16,), NEGINF, jnp.float32)
            return 0

        lax.fori_loop(0, (RB * C) // 128, init, 0)
        _process_bin(C, 128, vr, lo, bid_h, bdst_h, H_h,
                     histv, offsv, idw, dstw, hrows, hrows2, acc, sem, sem2)

        pltpu.sync_copy(batch_h.at[pl.ds(_m8(lo), RB)], batchv)

        def pinit(i, _):
            for u in range(8):
                pool[pl.ds(pl.multiple_of(i * 128 + u * 16, 16), 16)] = jnp.full((16,), NEGINF, jnp.float32)
            return 0

        lax.fori_loop(0, (GG * C) // 128, pinit, 0)
        nr = jnp.minimum(RB, NN - lo)

        def pnode(n, _):
            bv = batchv[pl.ds(pl.multiple_of((n >> 4) << 4, 16), 16)]
            bb = _lane_bcast(bv, n - ((n >> 4) << 4))
            a0 = bb * C
            for j in range(C // 16):
                aj = a0 + (iota + j * 16)
                v = jnp.maximum(acc[pl.ds(pl.multiple_of(n * C + j * 16, 16), 16)], 0.0)
                cur = plsc.load_gather(pool, [aj])
                plsc.store_scatter(pool, [aj], jnp.maximum(cur, v))
            return 0

        lax.fori_loop(0, nr, pnode, 0)
        pltpu.sync_copy(pool, part_h.at[pl.ds(_m8(vr * GG * C), GG * C)])


def _scat2_call(bid, bdst, hist, offs, H2, batch_p):
    return pl.kernel(
        _scat2_body,
        out_type=jax.ShapeDtypeStruct((NB * GG * 128,), jnp.float32),
        mesh=_mesh(),
        compiler_params=pltpu.CompilerParams(needs_layout_passes=False),
        scratch_types=[
            pltpu.VMEM((NW * NB,), jnp.int32),
            pltpu.VMEM((NW * NB,), jnp.int32),
            pltpu.VMEM((WIN,), jnp.int32),
            pltpu.VMEM((WIN,), jnp.int32),
            pltpu.VMEM((32, 128), jnp.float32),
            pltpu.VMEM((32, 128), jnp.float32),
            pltpu.VMEM((RB * 128,), jnp.float32),
            pltpu.VMEM((RB,), jnp.int32),
            pltpu.VMEM((GG * 128,), jnp.float32),
            pltpu.SemaphoreType.DMA,
            pltpu.SemaphoreType.DMA,
        ],
        name="sc_scat2",
    )(bid, bdst, hist, offs, H2, batch_p)


def _prep1_body(x_ref, W1_ref, b1_ref, A_ref, B_ref):
    Wt = W1_ref[0:4, :]
    Wb = W1_ref[4:8, :]
    x = x_ref[...]
    za = jnp.zeros((x.shape[0], 64), jnp.float32)
    A_ref[...] = jnp.concatenate(
        [jax.lax.dot_general(x, Wt - Wb, (((1,), (0,)), ((), ())),
                             preferred_element_type=jnp.float32) + b1_ref[...][None, :], za], axis=1)
    B_ref[...] = jnp.concatenate(
        [jax.lax.dot_general(x, Wb, (((1,), (0,)), ((), ())),
                             preferred_element_type=jnp.float32), za], axis=1)


def _prep1_call(x, W1, b1):
    return pl.pallas_call(
        _prep1_body,
        grid=(10,),
        in_specs=[
            pl.BlockSpec((5000, 4), lambda i: (i, 0)),
            pl.BlockSpec((8, 64), lambda i: (0, 0)),
            pl.BlockSpec((64,), lambda i: (0,)),
        ],
        out_specs=[
            pl.BlockSpec((5000, 128), lambda i: (i, 0)),
            pl.BlockSpec((5000, 128), lambda i: (i, 0)),
        ],
        out_shape=[
            jax.ShapeDtypeStruct((NN, 128), jnp.float32),
            jax.ShapeDtypeStruct((NN, 128), jnp.float32),
        ],
        name="tc_prep1",
    )(x, W1, b1)


def _prep2_body(h_ref, W3_ref, b3_ref, A_ref, B_ref):
    Wt = W3_ref[0:64, :]
    Wb = W3_ref[64:128, :]
    h = h_ref[...]
    A_ref[...] = jax.lax.dot_general(h, Wt - Wb, (((1,), (0,)), ((), ())),
                                     preferred_element_type=jnp.float32) + b3_ref[...][None, :]
    B_ref[...] = jax.lax.dot_general(h, Wb, (((1,), (0,)), ((), ())),
                                     preferred_element_type=jnp.float32)


def _prep2_call(h1, W3, b3):
    return pl.pallas_call(
        _prep2_body,
        grid=(10,),
        in_specs=[
            pl.BlockSpec((5000, 64), lambda i: (i, 0)),
            pl.BlockSpec((128, 128), lambda i: (0, 0)),
            pl.BlockSpec((128,), lambda i: (0,)),
        ],
        out_specs=[
            pl.BlockSpec((5000, 128), lambda i: (i, 0)),
            pl.BlockSpec((5000, 128), lambda i: (i, 0)),
        ],
        out_shape=[
            jax.ShapeDtypeStruct((NN, 128), jnp.float32),
            jax.ShapeDtypeStruct((NN, 128), jnp.float32),
        ],
        name="tc_prep2",
    )(h1, W3, b3)


def _mm_body(z_ref, W_ref, b_ref, out_ref):
    z = jnp.maximum(z_ref[...], 0.0)
    out_ref[...] = jax.lax.dot_general(z, W_ref[...], (((1,), (0,)), ((), ())),
                                       preferred_element_type=jnp.float32) + b_ref[...][None, :]


def _mm_call(C, Z, W, b):
    return pl.pallas_call(
        _mm_body,
        grid=(800,),
        in_specs=[
            pl.BlockSpec((1000, C), lambda i: (i, 0)),
            pl.BlockSpec((C, C), lambda i: (0, 0)),
            pl.BlockSpec((C,), lambda i: (0,)),
        ],
        out_specs=pl.BlockSpec((1000, C), lambda i: (i, 0)),
        out_shape=jax.ShapeDtypeStruct((EE, C), jnp.float32),
        name=f"tc_mm_{C}",
    )(Z, W, b)


def _head_body(part_ref, W5_ref, b5_ref, W6_ref, b6_ref, out_ref):
    pm = jnp.max(part_ref[...], axis=0)
    pooled = jnp.maximum(pm, 0.0)
    h = jnp.maximum(
        jax.lax.dot_general(pooled, W5_ref[...], (((1,), (0,)), ((), ())),
                            preferred_element_type=jnp.float32) + b5_ref[...][None, :], 0.0)
    o = jax.lax.dot_general(h, W6_ref[...], (((1,), (0,)), ((), ())),
                            preferred_element_type=jnp.float32) + b6_ref[...][None, :]
    m = jnp.max(o, axis=1, keepdims=True)
    lse = m + jnp.log(jnp.sum(jnp.exp(o - m), axis=1, keepdims=True))
    out_ref[...] = o - lse


def _head_call(part, W5, b5, W6, b6):
    return pl.pallas_call(
        _head_body,
        out_shape=jax.ShapeDtypeStruct((GG, 1), jnp.float32),
        name="tc_head",
    )(part, W5, b5, W6, b6)


def kernel(x, edge_index, batch, W1, b1, W2, b2, W3, b3, W4, b4, W5, b5, W6, b6):
    src = edge_index[0]
    dst = edge_index[1]
    dst_pad = jnp.concatenate([dst, jnp.zeros((16,), jnp.int32)])
    batch_pad = jnp.concatenate([batch, jnp.zeros((NB * RB - NN,), jnp.int32)])

    bid, bdst, hist, offs = _bin_call(dst_pad)

    A1, B1 = _prep1_call(x, W1, b1)
    Z1 = _edge_call(128, dst, src, A1, B1)
    W2p = jnp.zeros((128, 128), jnp.float32).at[:64, :64].set(W2)
    b2p = jnp.zeros((128,), jnp.float32).at[:64].set(b2)
    H1 = _mm_call(128, Z1, W2p, b2p)
    h1f = _scat1_call(bid, bdst, hist, offs, H1)
    h1 = h1f[: NN * 64].reshape(NN, 64)

    A2, B2 = _prep2_call(h1, W3, b3)
    Z2 = _edge_call(128, dst, src, A2, B2)
    H2 = _mm_call(128, Z2, W4, b4)
    part = _scat2_call(bid, bdst, hist, offs, H2, batch_pad)

    part3 = part.reshape(NB, GG, 128)
    return _head_call(part3, W5, b5, W6, b6)


# speedup vs baseline: 2.4433x; 1.1566x over previous
import functools

import jax
import jax.numpy as jnp
from jax import lax
from jax.experimental import pallas as pl
from jax.experimental.pallas import tpu as pltpu
from jax.experimental.pallas import tpu_sc as plsc

NN = 50000
EE = 800000
GG = 64
NC, NS, LN = 2, 16, 16
NW = NC * NS
EW = EE // NW
RB = 784
NB = 64
GST = 26624
LCAP = 25448
WIN = 1024
GBATCH = 64
NEGINF = float("-inf")


def _mesh():
    return plsc.VectorSubcoreMesh(core_axis_name="c", subcore_axis_name="s",
                                  num_cores=NC, num_subcores=NS)


def _wid():
    return lax.axis_index("c") * NS + lax.axis_index("s")


def _iota16():
    return lax.broadcasted_iota(jnp.int32, (16,), 0)


def _m8(v):
    return pl.multiple_of(v, 8)


def _scal(ref, i):
    base = pl.multiple_of((i >> 4) << 4, 16)
    v = ref[pl.ds(base, 16)]
    sel = jnp.where(_iota16() == (i - base), v, jnp.int32(-2147483647))
    return jnp.max(sel, axis=0)


def _lane_bcast(v, k):
    kv = jnp.zeros((16,), jnp.int32) + k
    return jnp.take_along_axis(v, kv, axis=0, mode="promise_in_bounds")


def _edge_body(CO, packed, dst_h, src_h, A_h, B_h, Z_h,
               idxd0, idxs0, idxd1, idxs1, ra0, rb0, ra1, rb1, zo0, zo1,
               sa0, sb0, sa1, sb1):
    w = _wid()
    base = w * EW
    bufs = ((idxd0, idxs0, ra0, rb0, zo0, sa0, sb0),
            (idxd1, idxs1, ra1, rb1, zo1, sa1, sb1))

    def start(i, bi):
        idxd, idxs, ra, rb, zo, sa, sb = bi
        off = _m8(base + i * 128)
        pltpu.sync_copy(dst_h.at[pl.ds(off, 128)], idxd)
        pltpu.sync_copy(src_h.at[pl.ds(off, 128)], idxs)
        pltpu.async_copy(A_h.at[idxd], ra, sa)
        pltpu.async_copy(B_h.at[idxs], rb, sb)

    def finish(i, bi):
        idxd, idxs, ra, rb, zo, sa, sb = bi
        off = _m8(base + i * 128)
        pltpu.make_async_copy(A_h.at[idxd], ra, sa).wait()
        pltpu.make_async_copy(B_h.at[idxs], rb, sb).wait()

        def add_body(r, _):
            for j in range(CO // 16):
                if packed:
                    zo[r, pl.ds(j * 16, 16)] = (
                        ra[r, pl.ds(j * 16, 16)] + rb[r, pl.ds(64 + j * 16, 16)]
                    )
                else:
                    ra[r, pl.ds(j * 16, 16)] = (
                        ra[r, pl.ds(j * 16, 16)] + rb[r, pl.ds(j * 16, 16)]
                    )
            return 0

        lax.fori_loop(0, 128, add_body, 0)
        out = zo if packed else ra
        pltpu.sync_copy(out, Z_h.at[pl.ds(off, 128)])

    start(0, bufs[0])

    def pair(g, _):
        for t in range(2):
            i = g * 2 + t

            @pl.when(i < 195)
            def _():
                @pl.when(i + 1 < 195)
                def _():
                    start(i + 1, bufs[(t + 1) % 2])

                finish(i, bufs[t])

        return 0

    lax.fori_loop(0, 98, pair, 0)

    idxd, idxs, ra, rb, zo, sa, sb = bufs[0]
    off = _m8(base + 195 * 128)
    pltpu.sync_copy(dst_h.at[pl.ds(off, 40)], idxd.at[pl.ds(0, 40)])
    pltpu.sync_copy(src_h.at[pl.ds(off, 40)], idxs.at[pl.ds(0, 40)])
    pltpu.async_copy(A_h.at[idxd.at[pl.ds(0, 40)]], ra.at[pl.ds(0, 40)], sa).wait()
    pltpu.async_copy(B_h.at[idxs.at[pl.ds(0, 40)]], rb.at[pl.ds(0, 40)], sb).wait()

    def tadd(r, _):
        for j in range(CO // 16):
            if packed:
                zo[r, pl.ds(j * 16, 16)] = (
                    ra[r, pl.ds(j * 16, 16)] + rb[r, pl.ds(64 + j * 16, 16)]
                )
            else:
                ra[r, pl.ds(j * 16, 16)] = (
                    ra[r, pl.ds(j * 16, 16)] + rb[r, pl.ds(j * 16, 16)]
                )
        return 0

    lax.fori_loop(0, 40, tadd, 0)
    out = zo if packed else ra
    pltpu.sync_copy(out.at[pl.ds(0, 40)], Z_h.at[pl.ds(off, 40)])


def _edge_call(CO, packed, dst, src, A, B):
    CT = 128
    body = functools.partial(_edge_body, CO, packed)
    return pl.kernel(
        body,
        out_type=jax.ShapeDtypeStruct((EE, CO), jnp.float32),
        mesh=_mesh(),
        compiler_params=pltpu.CompilerParams(needs_layout_passes=False),
        scratch_types=[
            pltpu.VMEM((128,), jnp.int32),
            pltpu.VMEM((128,), jnp.int32),
            pltpu.VMEM((128,), jnp.int32),
            pltpu.VMEM((128,), jnp.int32),
            pltpu.VMEM((128, CT), jnp.float32),
            pltpu.VMEM((128, CT), jnp.float32),
            pltpu.VMEM((128, CT), jnp.float32),
            pltpu.VMEM((128, CT), jnp.float32),
            pltpu.VMEM((128, CO), jnp.float32),
            pltpu.VMEM((128, CO), jnp.float32),
            pltpu.SemaphoreType.DMA,
            pltpu.SemaphoreType.DMA,
            pltpu.SemaphoreType.DMA,
            pltpu.SemaphoreType.DMA,
        ],
        name=f"sc_edge_{CO}",
    )(dst, src, A, B)


def _bin_body(dstp_h, bid_h, bdst_h, hist_h, offs_h, dstv, lid, ldst, hist, offs, rcnt):
    w = _wid()
    base = w * EW
    iota = _iota16()
    rcal, _ = plsc.scan_count(iota)
    bias = jnp.max(rcal, axis=0)

    for q in range(4):
        hist[pl.ds(q * 16, 16)] = jnp.zeros((16,), jnp.int32)

    def scan(pass2):
        def do_chunk(coff, sz, nv, tail):
            pltpu.sync_copy(dstp_h.at[pl.ds(_m8(base + coff), sz)], dstv.at[pl.ds(0, sz)])

            def vreg(j, _):
                d = dstv[pl.ds(j * 16, 16)]
                b = lax.div(d, jnp.int32(RB))
                if tail:
                    valid = (coff + j * 16 + iota) < EW
                else:
                    valid = jnp.full((16,), True)
                rank, lastm = plsc.scan_count(b, mask=valid)
                r0 = rank - bias
                if pass2:
                    cur = plsc.load_gather(rcnt, [b])
                    pos = cur + r0
                    eid = base + coff + j * 16 + iota
                    plsc.store_scatter(lid, [pos], eid, mask=valid)
                    plsc.store_scatter(ldst, [pos], d, mask=valid)
                    plsc.store_scatter(rcnt, [b], cur + r0 + 1, mask=lastm & valid)
                else:
                    cur = plsc.load_gather(hist, [b])
                    plsc.store_scatter(hist, [b], cur + r0 + 1, mask=lastm & valid)
                return 0

            lax.fori_loop(0, nv, vreg, 0)

        lax.fori_loop(0, 12, lambda i, _: (do_chunk(i * 2000, 2000, 125, False), 0)[1], 0)
        do_chunk(24000, 1008, 63, True)

    scan(False)

    carry = jnp.int32(0)
    for q in range(4):
        h16 = hist[pl.ds(q * 16, 16)]
        p16 = jnp.bitwise_and(h16 + 7, jnp.int32(-8))
        cs = plsc.cumsum(p16)
        offs[pl.ds(q * 16, 16)] = cs - p16 + carry
        carry = carry + jnp.max(cs, axis=0)
    for q in range(4):
        rcnt[pl.ds(q * 16, 16)] = offs[pl.ds(q * 16, 16)]

    scan(True)

    pltpu.sync_copy(hist, hist_h.at[pl.ds(_m8(w * NB), NB)])
    pltpu.sync_copy(offs, offs_h.at[pl.ds(_m8(w * NB), NB)])
    pltpu.sync_copy(lid, bid_h.at[pl.ds(_m8(w * GST), LCAP)])
    pltpu.sync_copy(ldst, bdst_h.at[pl.ds(_m8(w * GST), LCAP)])


def _bin_call(dstp):
    return pl.kernel(
        _bin_body,
        out_type=(
            jax.ShapeDtypeStruct((NW * GST,), jnp.int32),
            jax.ShapeDtypeStruct((NW * GST,), jnp.int32),
            jax.ShapeDtypeStruct((NW * NB,), jnp.int32),
            jax.ShapeDtypeStruct((NW * NB,), jnp.int32),
        ),
        mesh=_mesh(),
        compiler_params=pltpu.CompilerParams(needs_layout_passes=False),
        scratch_types=[
            pltpu.VMEM((2000,), jnp.int32),
            pltpu.VMEM((LCAP,), jnp.int32),
            pltpu.VMEM((LCAP,), jnp.int32),
            pltpu.VMEM((NB,), jnp.int32),
            pltpu.VMEM((NB,), jnp.int32),
            pltpu.VMEM((NB,), jnp.int32),
        ],
        name="sc_bin",
    )(dstp)


def _process_bin(C, CT, vr, lo, bid_h, bdst_h, H_h, histv, offsv, idw, dstw, hrows, hrows2, acc, sem, sem2):
    iota = _iota16()

    def per_wk(wk, _):
        cnt = _scal(histv, wk * NB + vr)
        off = _scal(offsv, wk * NB + vr)
        gbase = wk * GST + off
        nwin = (cnt + (WIN - 1)) // WIN

        def per_win(cw, _):
            vw = jnp.minimum(cnt - cw * WIN, WIN)
            pltpu.sync_copy(bid_h.at[pl.ds(_m8(gbase + cw * WIN), WIN)], idw)
            pltpu.sync_copy(bdst_h.at[pl.ds(_m8(gbase + cw * WIN), WIN)], dstw)
            safe = _scal(idw, 0)
            safed = _scal(dstw, 0)

            def fix(q, _):
                qb = pl.multiple_of(q * 16, 16)
                posv = q * 16 + iota
                idw[pl.ds(qb, 16)] = jnp.where(posv < vw, idw[pl.ds(qb, 16)], safe)
                dstw[pl.ds(qb, 16)] = jnp.where(posv < vw, dstw[pl.ds(qb, 16)], safed)
                return 0

            lax.fori_loop(0, WIN // 16, fix, 0)
            gb = hrows.shape[0]
            nb = (vw + (gb - 1)) // gb
            pltpu.async_copy(H_h.at[idw.at[pl.ds(0, gb)]], hrows, sem)

            def do_batch(b, hr, sm, hrn, smn):
                gb = hr.shape[0]

                @pl.when(b + 1 < nb)
                def _():
                    pltpu.async_copy(
                        H_h.at[idw.at[pl.ds(_m8((b + 1) * gb), gb)]], hrn, smn
                    )

                pltpu.make_async_copy(H_h.at[idw.at[pl.ds(0, gb)]], hr, sm).wait()
                ne = jnp.minimum(vw - b * gb, gb)

                def per_pair(g, _):
                    for t in range(2):
                        e = g * 2 + t
                        q = b * gb + e
                        dv = dstw[pl.ds(pl.multiple_of((q >> 4) << 4, 16), 16)]
                        db = _lane_bcast(dv, q - ((q >> 4) << 4))
                        a0 = (db - lo) * C
                        for j in range(C // 16):
                            aj = a0 + (iota + j * 16)
                            cur = plsc.load_gather(acc, [aj])
                            hv = hr[e, pl.ds(j * 16, 16)]
                            plsc.store_scatter(acc, [aj], jnp.maximum(cur, hv))
                    return 0

                lax.fori_loop(0, (ne + 1) >> 1, per_pair, 0)

            def per_bpair(gb, _):
                @pl.when(2 * gb < nb)
                def _():
                    do_batch(2 * gb, hrows, sem, hrows2, sem2)

                @pl.when(2 * gb + 1 < nb)
                def _():
                    do_batch(2 * gb + 1, hrows2, sem2, hrows, sem)

                return 0

            lax.fori_loop(0, (nb + 1) >> 1, per_bpair, 0)
            return 0

        lax.fori_loop(0, nwin, per_win, 0)
        return 0

    lax.fori_loop(0, NW, per_wk, 0)


def _scat1_body(bid_h, bdst_h, hist_h, offs_h, H_h, h1_h,
                histv, offsv, idw, dstw, hrows, hrows2, acc, sem, sem2):
    C = 64
    w = _wid()
    lo = w * (2 * RB)
    pltpu.sync_copy(hist_h, histv)
    pltpu.sync_copy(offs_h, offsv)

    def init(i, _):
        for u in range(8):
            acc[pl.ds(pl.multiple_of(i * 128 + u * 16, 16), 16)] = jnp.full((16,), NEGINF, jnp.float32)
        return 0

    lax.fori_loop(0, (2 * RB * C) // 128, init, 0)
    for sb in range(2):
        _process_bin(C, 128, 2 * w + sb, lo, bid_h, bdst_h, H_h,
                     histv, offsv, idw, dstw, hrows, hrows2, acc, sem, sem2)

    def fin(i, _):
        for u in range(8):
            j = pl.multiple_of(i * 128 + u * 16, 16)
            acc[pl.ds(j, 16)] = jnp.maximum(acc[pl.ds(j, 16)], 0.0)
        return 0

    lax.fori_loop(0, (2 * RB * C) // 128, fin, 0)
    pltpu.sync_copy(acc, h1_h.at[pl.ds(_m8(lo * C), 2 * RB * C)])


def _scat1_call(bid, bdst, hist, offs, H1):
    return pl.kernel(
        _scat1_body,
        out_type=jax.ShapeDtypeStruct((NW * 2 * RB * 64,), jnp.float32),
        mesh=_mesh(),
        compiler_params=pltpu.CompilerParams(needs_layout_passes=False),
        scratch_types=[
            pltpu.VMEM((NW * NB,), jnp.int32),
            pltpu.VMEM((NW * NB,), jnp.int32),
            pltpu.VMEM((WIN,), jnp.int32),
            pltpu.VMEM((WIN,), jnp.int32),
            pltpu.VMEM((GBATCH, 128), jnp.float32),
            pltpu.VMEM((GBATCH, 128), jnp.float32),
            pltpu.VMEM((2 * RB * 64,), jnp.float32),
            pltpu.SemaphoreType.DMA,
            pltpu.SemaphoreType.DMA,
        ],
        name="sc_scat1",
    )(bid, bdst, hist, offs, H1)


def _scat2_body(bid_h, bdst_h, hist_h, offs_h, H_h, batch_h, part_h,
                histv, offsv, idw, dstw, hrows, hrows2, acc, batchv, pool, sem, sem2):
    C = 128
    w = _wid()
    iota = _iota16()
    pltpu.sync_copy(hist_h, histv)
    pltpu.sync_copy(offs_h, offsv)
    for sb in range(2):
        vr = 2 * w + sb
        lo = vr * RB

        def init(i, _):
            for u in range(8):
                acc[pl.ds(pl.multiple_of(i * 128 + u * 16, 16), 16)] = jnp.full((16,), NEGINF, jnp.float32)
            return 0

        lax.fori_loop(0, (RB * C) // 128, init, 0)
        _process_bin(C, 128, vr, lo, bid_h, bdst_h, H_h,
                     histv, offsv, idw, dstw, hrows, hrows2, acc, sem, sem2)

        pltpu.sync_copy(batch_h.at[pl.ds(_m8(lo), RB)], batchv)

        def pinit(i, _):
            for u in range(8):
                pool[pl.ds(pl.multiple_of(i * 128 + u * 16, 16), 16)] = jnp.full((16,), NEGINF, jnp.float32)
            return 0

        lax.fori_loop(0, (GG * C) // 128, pinit, 0)
        nr = jnp.minimum(RB, NN - lo)

        def pnode(n, _):
            bv = batchv[pl.ds(pl.multiple_of((n >> 4) << 4, 16), 16)]
            bb = _lane_bcast(bv, n - ((n >> 4) << 4))
            a0 = bb * C
            for j in range(C // 16):
                aj = a0 + (iota + j * 16)
                v = jnp.maximum(acc[pl.ds(pl.multiple_of(n * C + j * 16, 16), 16)], 0.0)
                cur = plsc.load_gather(pool, [aj])
                plsc.store_scatter(pool, [aj], jnp.maximum(cur, v))
            return 0

        lax.fori_loop(0, nr, pnode, 0)
        pltpu.sync_copy(pool, part_h.at[pl.ds(_m8(vr * GG * C), GG * C)])


def _scat2_call(bid, bdst, hist, offs, H2, batch_p):
    return pl.kernel(
        _scat2_body,
        out_type=jax.ShapeDtypeStruct((NB * GG * 128,), jnp.float32),
        mesh=_mesh(),
        compiler_params=pltpu.CompilerParams(needs_layout_passes=False),
        scratch_types=[
            pltpu.VMEM((NW * NB,), jnp.int32),
            pltpu.VMEM((NW * NB,), jnp.int32),
            pltpu.VMEM((WIN,), jnp.int32),
            pltpu.VMEM((WIN,), jnp.int32),
            pltpu.VMEM((32, 128), jnp.float32),
            pltpu.VMEM((32, 128), jnp.float32),
            pltpu.VMEM((RB * 128,), jnp.float32),
            pltpu.VMEM((RB,), jnp.int32),
            pltpu.VMEM((GG * 128,), jnp.float32),
            pltpu.SemaphoreType.DMA,
            pltpu.SemaphoreType.DMA,
        ],
        name="sc_scat2",
    )(bid, bdst, hist, offs, H2, batch_p)


def _prep1_body(x_ref, W1_ref, b1_ref, T_ref):
    Wt = W1_ref[0:4, :]
    Wb = W1_ref[4:8, :]
    x = x_ref[...]
    A = jax.lax.dot_general(x, Wt - Wb, (((1,), (0,)), ((), ())),
                            preferred_element_type=jnp.float32) + b1_ref[...][None, :]
    B = jax.lax.dot_general(x, Wb, (((1,), (0,)), ((), ())),
                            preferred_element_type=jnp.float32)
    T_ref[...] = jnp.concatenate([A, B], axis=1)


def _prep1_call(x, W1, b1):
    return pl.pallas_call(
        _prep1_body,
        grid=(10,),
        in_specs=[
            pl.BlockSpec((5000, 4), lambda i: (i, 0)),
            pl.BlockSpec((8, 64), lambda i: (0, 0)),
            pl.BlockSpec((64,), lambda i: (0,)),
        ],
        out_specs=pl.BlockSpec((5000, 128), lambda i: (i, 0)),
        out_shape=jax.ShapeDtypeStruct((NN, 128), jnp.float32),
        name="tc_prep1",
    )(x, W1, b1)


def _prep2_body(h_ref, W3_ref, b3_ref, A_ref, B_ref):
    Wt = W3_ref[0:64, :]
    Wb = W3_ref[64:128, :]
    h = h_ref[...]
    A_ref[...] = jax.lax.dot_general(h, Wt - Wb, (((1,), (0,)), ((), ())),
                                     preferred_element_type=jnp.float32) + b3_ref[...][None, :]
    B_ref[...] = jax.lax.dot_general(h, Wb, (((1,), (0,)), ((), ())),
                                     preferred_element_type=jnp.float32)


def _prep2_call(h1, W3, b3):
    return pl.pallas_call(
        _prep2_body,
        grid=(10,),
        in_specs=[
            pl.BlockSpec((5000, 64), lambda i: (i, 0)),
            pl.BlockSpec((128, 128), lambda i: (0, 0)),
            pl.BlockSpec((128,), lambda i: (0,)),
        ],
        out_specs=[
            pl.BlockSpec((5000, 128), lambda i: (i, 0)),
            pl.BlockSpec((5000, 128), lambda i: (i, 0)),
        ],
        out_shape=[
            jax.ShapeDtypeStruct((NN, 128), jnp.float32),
            jax.ShapeDtypeStruct((NN, 128), jnp.float32),
        ],
        name="tc_prep2",
    )(h1, W3, b3)


def _mm_body(z_ref, W_ref, b_ref, out_ref):
    z = jnp.maximum(z_ref[...], 0.0)
    out_ref[...] = jax.lax.dot_general(z, W_ref[...], (((1,), (0,)), ((), ())),
                                       preferred_element_type=jnp.float32) + b_ref[...][None, :]


def _mm_call(CIN, COUT, Z, W, b):
    return pl.pallas_call(
        _mm_body,
        grid=(800,),
        in_specs=[
            pl.BlockSpec((1000, CIN), lambda i: (i, 0)),
            pl.BlockSpec((CIN, COUT), lambda i: (0, 0)),
            pl.BlockSpec((COUT,), lambda i: (0,)),
        ],
        out_specs=pl.BlockSpec((1000, COUT), lambda i: (i, 0)),
        out_shape=jax.ShapeDtypeStruct((EE, COUT), jnp.float32),
        name=f"tc_mm_{CIN}_{COUT}",
    )(Z, W, b)


def _head_body(part_ref, W5_ref, b5_ref, W6_ref, b6_ref, out_ref):
    pm = jnp.max(part_ref[...], axis=0)
    pooled = jnp.maximum(pm, 0.0)
    h = jnp.maximum(
        jax.lax.dot_general(pooled, W5_ref[...], (((1,), (0,)), ((), ())),
                            preferred_element_type=jnp.float32) + b5_ref[...][None, :], 0.0)
    o = jax.lax.dot_general(h, W6_ref[...], (((1,), (0,)), ((), ())),
                            preferred_element_type=jnp.float32) + b6_ref[...][None, :]
    m = jnp.max(o, axis=1, keepdims=True)
    lse = m + jnp.log(jnp.sum(jnp.exp(o - m), axis=1, keepdims=True))
    out_ref[...] = o - lse


def _head_call(part, W5, b5, W6, b6):
    return pl.pallas_call(
        _head_body,
        out_shape=jax.ShapeDtypeStruct((GG, 1), jnp.float32),
        name="tc_head",
    )(part, W5, b5, W6, b6)


def kernel(x, edge_index, batch, W1, b1, W2, b2, W3, b3, W4, b4, W5, b5, W6, b6):
    src = edge_index[0]
    dst = edge_index[1]
    dst_pad = jnp.concatenate([dst, jnp.zeros((16,), jnp.int32)])
    batch_pad = jnp.concatenate([batch, jnp.zeros((NB * RB - NN,), jnp.int32)])

    bid, bdst, hist, offs = _bin_call(dst_pad)

    T1 = _prep1_call(x, W1, b1)
    Z1 = _edge_call(64, True, dst, src, T1, T1)
    W2p = jnp.zeros((64, 128), jnp.float32).at[:, :64].set(W2)
    b2p = jnp.zeros((128,), jnp.float32).at[:64].set(b2)
    H1 = _mm_call(64, 128, Z1, W2p, b2p)
    h1f = _scat1_call(bid, bdst, hist, offs, H1)
    h1 = h1f[: NN * 64].reshape(NN, 64)

    A2, B2 = _prep2_call(h1, W3, b3)
    Z2 = _edge_call(128, False, dst, src, A2, B2)
    H2 = _mm_call(128, 128, Z2, W4, b4)
    part = _scat2_call(bid, bdst, hist, offs, H2, batch_pad)

    part3 = part.reshape(NB, GG, 128)
    return _head_call(part3, W5, b5, W6, b6)
